# baseline (speedup 1.0000x reference)
import functools

import jax
import jax.numpy as jnp
from jax import lax
from jax.experimental import pallas as pl
from jax.experimental.pallas import tpu as pltpu
from jax.experimental.pallas import tpu_sc as plsc

NC = 2
NS = 16
NW = NC * NS
CHUNK = 128


def kernel(condition, learned_embs, uncond_embedding):
    B = condition.shape[0]
    V, D = learned_embs.shape
    b_per_w = B // NW
    n_chunks = b_per_w // CHUNK

    idx = condition.astype(jnp.int32).reshape(NW, n_chunks, CHUNK)
    mesh = plsc.VectorSubcoreMesh(core_axis_name="c", subcore_axis_name="s")

    @functools.partial(
        pl.kernel,
        mesh=mesh,
        out_type=jax.ShapeDtypeStruct((B, D), jnp.float32),
        scratch_types=[
            pltpu.VMEM((n_chunks, CHUNK), jnp.int32),
            pltpu.VMEM((b_per_w, D), jnp.float32),
            pltpu.SemaphoreType.DMA,
        ],
        compiler_params=pltpu.CompilerParams(use_tc_tiling_on_sc=False),
    )
    def emb_gather(idx_hbm, table_hbm, out_hbm, idx_v, rows_v, sem):
        wid = lax.axis_index("s") * NC + lax.axis_index("c")
        pltpu.sync_copy(idx_hbm.at[wid], idx_v)
        copies = [
            pltpu.async_copy(
                table_hbm.at[idx_v.at[j]],
                rows_v.at[pl.ds(j * CHUNK, CHUNK)],
                sem,
            )
            for j in range(n_chunks)
        ]
        for c in copies:
            c.wait()
        pltpu.sync_copy(rows_v, out_hbm.at[pl.ds(wid * b_per_w, b_per_w)])

    return emb_gather(idx, learned_embs)


# speedup vs baseline: 2.1744x; 2.1744x over previous
import functools

import jax
import jax.numpy as jnp
from jax import lax
from jax.experimental import pallas as pl
from jax.experimental.pallas import tpu as pltpu
from jax.experimental.pallas import tpu_sc as plsc

NC = 2
NS = 16
NW = NC * NS
CHUNK = 128
LBLK = 8192


def _reformat_body(tin_ref, z_ref):
    z_ref[:, 0:64] = tin_ref[:, 0:4096].T
    z_ref[:, 64:128] = tin_ref[:, 4096:8192].T


def _tc_reformat(tbl_t):
    D, V = tbl_t.shape
    grid = (V + LBLK - 1) // LBLK
    return pl.pallas_call(
        _reformat_body,
        grid=(grid,),
        in_specs=[pl.BlockSpec((D, LBLK), lambda j: (0, j))],
        out_specs=pl.BlockSpec((LBLK // 2, 128), lambda j: (j, 0)),
        out_shape=jax.ShapeDtypeStruct((grid * (LBLK // 2), 128), jnp.float32),
    )(tbl_t)


def kernel(condition, learned_embs, uncond_embedding):
    B = condition.shape[0]
    V, D = learned_embs.shape
    b_per_w = B // NW
    n_chunks = b_per_w // CHUNK

    idx = condition.astype(jnp.int32).reshape(NW, n_chunks, CHUNK)
    z = _tc_reformat(learned_embs.T)
    z64 = z.reshape(z.shape[0] * 2, D)

    mesh = plsc.VectorSubcoreMesh(core_axis_name="c", subcore_axis_name="s")

    @functools.partial(
        pl.kernel,
        mesh=mesh,
        out_type=jax.ShapeDtypeStruct((B, D), jnp.float32),
        scratch_types=[
            pltpu.VMEM((n_chunks, CHUNK), jnp.int32),
            pltpu.VMEM((n_chunks, CHUNK), jnp.int32),
            pltpu.VMEM((b_per_w, D), jnp.float32),
            pltpu.SemaphoreType.DMA,
        ],
        compiler_params=pltpu.CompilerParams(use_tc_tiling_on_sc=False),
    )
    def emb_gather(idx_hbm, table_hbm, out_hbm, idx_v, row_v, rows_v, sem):
        wid = lax.axis_index("s") * NC + lax.axis_index("c")
        pltpu.sync_copy(idx_hbm.at[wid], idx_v)
        for j in range(n_chunks):
            for k in range(CHUNK // 16):
                r = idx_v[j, pl.ds(k * 16, 16)]
                rr = ((r >> 13) << 13) + ((r & 4095) << 1) + ((r >> 12) & 1)
                row_v[j, pl.ds(k * 16, 16)] = rr
        copies = [
            pltpu.async_copy(
                table_hbm.at[row_v.at[j]],
                rows_v.at[pl.ds(j * CHUNK, CHUNK)],
                sem,
            )
            for j in range(n_chunks)
        ]
        for cp in copies:
            cp.wait()
        pltpu.sync_copy(rows_v, out_hbm.at[pl.ds(wid * b_per_w, b_per_w)])

    return emb_gather(idx, z64)


# speedup vs baseline: 2.7765x; 1.2769x over previous
import functools

import jax
import jax.numpy as jnp
from jax import lax
from jax.experimental import pallas as pl
from jax.experimental.pallas import tpu as pltpu
from jax.experimental.pallas import tpu_sc as plsc

NC = 2
NS = 16
NW = NC * NS
CHUNK = 128
LBLK = 8192


def _reformat_body(tin_ref, z_ref):
    s = jnp.concatenate([tin_ref[:, 0:4096], tin_ref[:, 4096:8192]], axis=0)
    z_ref[...] = s.T


def _tc_reformat(tbl_t):
    D, V = tbl_t.shape
    grid = (V + LBLK - 1) // LBLK
    return pl.pallas_call(
        _reformat_body,
        grid=(grid,),
        in_specs=[pl.BlockSpec((D, LBLK), lambda j: (0, j))],
        out_specs=pl.BlockSpec((LBLK // 2, 128), lambda j: (j, 0)),
        out_shape=jax.ShapeDtypeStruct((grid * (LBLK // 2), 128), jnp.float32),
    )(tbl_t)


def kernel(condition, learned_embs, uncond_embedding):
    B = condition.shape[0]
    V, D = learned_embs.shape
    b_per_w = B // NW
    n_chunks = b_per_w // CHUNK

    idx = condition.astype(jnp.int32).reshape(NW, n_chunks, CHUNK)
    z = _tc_reformat(learned_embs.T)
    z64 = z.reshape(z.shape[0] * 2, D)

    mesh = plsc.VectorSubcoreMesh(core_axis_name="c", subcore_axis_name="s")

    @functools.partial(
        pl.kernel,
        mesh=mesh,
        out_type=jax.ShapeDtypeStruct((B, D), jnp.float32),
        scratch_types=[
            pltpu.VMEM((n_chunks, CHUNK), jnp.int32),
            pltpu.VMEM((n_chunks, CHUNK), jnp.int32),
            pltpu.VMEM((b_per_w, D), jnp.float32),
            pltpu.SemaphoreType.DMA,
        ],
        compiler_params=pltpu.CompilerParams(use_tc_tiling_on_sc=False),
    )
    def emb_gather(idx_hbm, table_hbm, out_hbm, idx_v, row_v, rows_v, sem):
        wid = lax.axis_index("s") * NC + lax.axis_index("c")
        pltpu.sync_copy(idx_hbm.at[wid], idx_v)
        for j in range(n_chunks):
            for k in range(CHUNK // 16):
                r = idx_v[j, pl.ds(k * 16, 16)]
                rr = ((r >> 13) << 13) + ((r & 4095) << 1) + ((r >> 12) & 1)
                row_v[j, pl.ds(k * 16, 16)] = rr
        copies = [
            pltpu.async_copy(
                table_hbm.at[row_v.at[j]],
                rows_v.at[pl.ds(j * CHUNK, CHUNK)],
                sem,
            )
            for j in range(n_chunks)
        ]
        for cp in copies:
            cp.wait()
        pltpu.sync_copy(rows_v, out_hbm.at[pl.ds(wid * b_per_w, b_per_w)])

    return emb_gather(idx, z64)
